# Initial kernel scaffold; baseline (speedup 1.0000x reference)
import jax
import jax.numpy as jnp
from jax.experimental import pallas as pl


def kernel(x, edge_index, edge_weight, labels, alpha1, alpha2, alpha3, alpha4):
    raise NotImplementedError("write your pallas kernel here")


# speedup vs baseline: 15.4088x; 15.4088x over previous
import functools

import jax
import jax.numpy as jnp
from jax import lax
from jax.experimental import pallas as pl
from jax.experimental.pallas import tpu as pltpu
from jax.experimental.pallas import tpu_sc as plsc

NC = 2
NS = 16
NW = NC * NS
L = 16
BCH = 1024
CH = 512

_mesh = functools.partial(
    plsc.VectorSubcoreMesh, core_axis_name="c", subcore_axis_name="s"
)


def _worker_id():
    return lax.axis_index("s") * NC + lax.axis_index("c")


@functools.lru_cache(maxsize=None)
def _build(N, D, E):
    N1 = ((N + 1 + NW * L - 1) // (NW * L)) * (NW * L)
    NCHB = (E + NW * BCH - 1) // (NW * BCH)
    EP = NW * BCH * NCHB
    NCH = EP // (NW * CH)
    ER = EP // 128
    SL = N1 // NS
    RPT = N1 // NW
    f32 = jnp.float32
    i32 = jnp.int32

    @functools.partial(
        pl.kernel,
        out_type=jax.ShapeDtypeStruct((NC * N1,), f32),
        mesh=_mesh(),
        compiler_params=pltpu.CompilerParams(use_tc_tiling_on_sc=False),
        scratch_types=[
            pltpu.VMEM((8, 128), f32),
            pltpu.VMEM((8, 128), i32),
            pltpu.VMEM((BCH,), f32),
            pltpu.VMEM((L,), f32),
            pltpu.VMEM_SHARED((N1,), f32),
            pltpu.SemaphoreType.DMA,
        ],
    )
    def bias_partial(w_h, dst_h, a3_h, bp_h, wbuf, dbuf, vbuf, abuf, bacc, sem):
        c = lax.axis_index("c")
        s = lax.axis_index("s")
        w = _worker_id()
        sync = pltpu.sync_copy
        for k in range(BCH // L):
            vbuf[pl.ds(k * L, L)] = jnp.zeros((L,), f32)
        base = s * SL
        nfull = SL // BCH
        for t in range(nfull):
            sync(vbuf, bacc.at[pl.ds(base + t * BCH, BCH)])
        rem = SL - nfull * BCH
        if rem:
            sync(vbuf.at[pl.ds(0, rem)], bacc.at[pl.ds(base + nfull * BCH, rem)])
        sync(a3_h, abuf)
        plsc.subcore_barrier()
        a3 = abuf[...]

        def chunk(j, carry):
            r0 = (w * NCHB + j) * 8
            sync(w_h.at[pl.ds(r0, 8)], wbuf)
            sync(dst_h.at[pl.ds(r0, 8)], dbuf)
            for r in range(8):
                for k in range(8):
                    v = wbuf[r, pl.ds(k * L, L)]
                    vbuf[pl.ds(r * 128 + k * L, L)] = jnp.maximum(a3 * v, 0.0)
            descs = [
                pltpu.async_copy(
                    vbuf.at[pl.ds(r * 128, 128)], bacc.at[dbuf.at[r]], sem, add=True
                )
                for r in range(8)
            ]
            for dsc in descs:
                dsc.wait()
            return carry

        lax.fori_loop(0, NCHB, chunk, 0)
        plsc.subcore_barrier()
        for t in range(nfull):
            sync(bacc.at[pl.ds(base + t * BCH, BCH)], vbuf)
            sync(vbuf, bp_h.at[pl.ds(c * N1 + base + t * BCH, BCH)])
        if rem:
            sync(bacc.at[pl.ds(base + nfull * BCH, rem)], vbuf.at[pl.ds(0, rem)])
            sync(vbuf.at[pl.ds(0, rem)], bp_h.at[pl.ds(c * N1 + base + nfull * BCH, rem)])

    @functools.partial(
        pl.kernel,
        out_type=jax.ShapeDtypeStruct((N1, D), f32),
        mesh=_mesh(),
        compiler_params=pltpu.CompilerParams(use_tc_tiling_on_sc=False),
        scratch_types=[
            pltpu.VMEM((RPT,), f32),
            pltpu.VMEM((RPT,), f32),
            pltpu.VMEM((RPT,), f32),
            pltpu.VMEM((L,), f32),
            pltpu.VMEM((L,), f32),
            pltpu.VMEM((RPT, D), f32),
        ],
    )
    def bias_rep(bp_h, lab_h, a2_h, a4_h, brep_h, b0, b1, lb, a2b, a4b, ob):
        w = _worker_id()
        base = w * RPT
        sync = pltpu.sync_copy
        sync(bp_h.at[pl.ds(base, RPT)], b0)
        sync(bp_h.at[pl.ds(N1 + base, RPT)], b1)
        sync(lab_h.at[pl.ds(base, RPT)], lb)
        sync(a2_h, a2b)
        sync(a4_h, a4b)
        a2 = a2b[...]
        a4 = a4b[...]

        def red(i, carry):
            o = i * L
            b0[pl.ds(o, L)] = a2 * (b0[pl.ds(o, L)] + b1[pl.ds(o, L)]) + a4 * lb[pl.ds(o, L)]
            return carry

        lax.fori_loop(0, RPT // L, red, 0)

        def bcast(i, carry):
            t16 = b0[pl.ds(i * L, L)]
            for k in range(L):
                t = jnp.full((L,), t16[k], f32)
                r = i * L + k
                for h in range(D // L):
                    ob[r, pl.ds(h * L, L)] = t
            return carry

        lax.fori_loop(0, RPT // L, bcast, 0)
        sync(ob, brep_h.at[pl.ds(base, RPT), :])

    @functools.partial(
        pl.kernel,
        out_type=jax.ShapeDtypeStruct((NC * N1, D), f32),
        mesh=_mesh(),
        compiler_params=pltpu.CompilerParams(use_tc_tiling_on_sc=False),
        scratch_types=[
            pltpu.VMEM((CH // 128, 128), i32),
            pltpu.VMEM((CH // 128, 128), i32),
            pltpu.VMEM((CH, D), f32),
            pltpu.VMEM_SHARED((N1, D), f32),
            pltpu.SemaphoreType.DMA,
            pltpu.SemaphoreType.DMA,
        ],
    )
    def scatter_k(y_h, src_h, dst_h, pp_h, sidx, didx, rows, acc, gsem, ssem):
        c = lax.axis_index("c")
        s = lax.axis_index("s")
        w = _worker_id()
        sync = pltpu.sync_copy

        def zr(r, carry):
            for h in range(D // L):
                rows[r, pl.ds(h * L, L)] = jnp.zeros((L,), f32)
            return carry

        lax.fori_loop(0, CH, zr, 0)
        base = s * SL
        nfull = SL // CH
        for t in range(nfull):
            sync(rows, acc.at[pl.ds(base + t * CH, CH), :])
        rem = SL - nfull * CH
        if rem:
            sync(rows.at[pl.ds(0, rem)], acc.at[pl.ds(base + nfull * CH, rem), :])
        plsc.subcore_barrier()

        NB = CH // 128

        def chunk(j, carry):
            r0 = (w * NCH + j) * NB
            sync(src_h.at[pl.ds(r0, NB)], sidx)
            sync(dst_h.at[pl.ds(r0, NB)], didx)
            g = [
                pltpu.async_copy(
                    y_h.at[sidx.at[b]], rows.at[pl.ds(b * 128, 128)], gsem
                )
                for b in range(NB)
            ]
            for dsc in g:
                dsc.wait()
            sc = [
                pltpu.async_copy(
                    rows.at[pl.ds(b * 128, 128)], acc.at[didx.at[b]], ssem, add=True
                )
                for b in range(NB)
            ]
            for dsc in sc:
                dsc.wait()
            return carry

        lax.fori_loop(0, NCH, chunk, 0)
        plsc.subcore_barrier()
        for t in range(nfull):
            sync(acc.at[pl.ds(base + t * CH, CH), :], rows)
            sync(rows, pp_h.at[pl.ds(c * N1 + base + t * CH, CH), :])
        if rem:
            sync(acc.at[pl.ds(base + nfull * CH, rem), :], rows.at[pl.ds(0, rem)])
            sync(rows.at[pl.ds(0, rem)], pp_h.at[pl.ds(c * N1 + base + nfull * CH, rem), :])

    CB = 512

    @functools.partial(
        pl.kernel,
        out_type=jax.ShapeDtypeStruct((N1, D), f32),
        mesh=_mesh(),
        compiler_params=pltpu.CompilerParams(use_tc_tiling_on_sc=False),
        scratch_types=[
            pltpu.VMEM((CB, D), f32),
            pltpu.VMEM((CB, D), f32),
            pltpu.VMEM((CB, D), f32),
            pltpu.VMEM((CB, D), f32),
            pltpu.VMEM((L,), f32),
        ],
    )
    def combine_k(pp_h, brep_h, a1_h, y_h, p0, p1, bb, ob, a1b):
        w = _worker_id()
        base = w * RPT
        sync = pltpu.sync_copy
        sync(a1_h, a1b)
        a1 = a1b[...]
        offs = [(o, min(CB, RPT - o)) for o in range(0, RPT, CB)]
        for o, n in offs:
            sync(pp_h.at[pl.ds(base + o, n), :], p0.at[pl.ds(0, n)])
            sync(pp_h.at[pl.ds(N1 + base + o, n), :], p1.at[pl.ds(0, n)])
            sync(brep_h.at[pl.ds(base + o, n), :], bb.at[pl.ds(0, n)])

            def row(r, carry):
                for h in range(D // L):
                    hs = pl.ds(h * L, L)
                    v = a1 * (p0[r, hs] + p1[r, hs]) + bb[r, hs]
                    ob[r, hs] = jnp.maximum(v, 0.0)
                return carry

            lax.fori_loop(0, n, row, 0)
            sync(ob.at[pl.ds(0, n)], y_h.at[pl.ds(base + o, n), :])

    return bias_partial, bias_rep, scatter_k, combine_k, N1, EP


def kernel(x, edge_index, edge_weight, labels, alpha1, alpha2, alpha3, alpha4):
    N, D = x.shape
    E = edge_index.shape[1]
    bias_partial, bias_rep, scatter_k, combine_k, N1, EP = _build(N, D, E)
    f32 = jnp.float32
    i32 = jnp.int32

    pe = EP - E
    src2 = jnp.concatenate([edge_index[0], jnp.zeros((pe,), i32)]).reshape(EP // 128, 128)
    dst2 = jnp.concatenate([edge_index[1], jnp.full((pe,), N, i32)]).reshape(EP // 128, 128)
    w2 = jnp.concatenate([edge_weight.astype(f32), jnp.zeros((pe,), f32)]).reshape(EP // 128, 128)
    xp = jnp.concatenate([x.astype(f32), jnp.zeros((N1 - N, D), f32)], axis=0)
    labp = jnp.concatenate([labels.astype(f32), jnp.zeros((N1 - N,), f32)])
    a1v = jnp.broadcast_to(alpha1.astype(f32), (L,))
    a2v = jnp.broadcast_to(alpha2.astype(f32), (L,))
    a3v = jnp.broadcast_to(alpha3.astype(f32), (L,))
    a4v = jnp.broadcast_to(alpha4.astype(f32), (L,))

    bp = bias_partial(w2, dst2, a3v)
    brep = bias_rep(bp, labp, a2v, a4v)
    y = xp
    for _ in range(4):
        pp = scatter_k(y, src2, dst2)
        y = combine_k(pp, brep, a1v)
    return y[:N]


# speedup vs baseline: 17.2532x; 1.1197x over previous
import functools

import jax
import jax.numpy as jnp
from jax import lax
from jax.experimental import pallas as pl
from jax.experimental.pallas import tpu as pltpu
from jax.experimental.pallas import tpu_sc as plsc

NC = 2
NS = 16
NW = NC * NS
L = 16
BCH = 1024
CH = 256

_mesh = functools.partial(
    plsc.VectorSubcoreMesh, core_axis_name="c", subcore_axis_name="s"
)


def _worker_id():
    return lax.axis_index("s") * NC + lax.axis_index("c")


@functools.lru_cache(maxsize=None)
def _build(N, D, E):
    N1 = ((N + 1 + NW * L - 1) // (NW * L)) * (NW * L)
    NCHB = (E + NW * BCH - 1) // (NW * BCH)
    EP = NW * BCH * NCHB
    NCH = EP // (NW * CH)
    ER = EP // 128
    SL = N1 // NS
    RPT = N1 // NW
    f32 = jnp.float32
    i32 = jnp.int32

    @functools.partial(
        pl.kernel,
        out_type=jax.ShapeDtypeStruct((NC * N1,), f32),
        mesh=_mesh(),
        compiler_params=pltpu.CompilerParams(use_tc_tiling_on_sc=False),
        scratch_types=[
            pltpu.VMEM((8, 128), f32),
            pltpu.VMEM((8, 128), i32),
            pltpu.VMEM((BCH,), f32),
            pltpu.VMEM((L,), f32),
            pltpu.VMEM_SHARED((N1,), f32),
            pltpu.SemaphoreType.DMA,
        ],
    )
    def bias_partial(w_h, dst_h, a3_h, bp_h, wbuf, dbuf, vbuf, abuf, bacc, sem):
        c = lax.axis_index("c")
        s = lax.axis_index("s")
        w = _worker_id()
        sync = pltpu.sync_copy
        for k in range(BCH // L):
            vbuf[pl.ds(k * L, L)] = jnp.zeros((L,), f32)
        base = s * SL
        nfull = SL // BCH
        for t in range(nfull):
            sync(vbuf, bacc.at[pl.ds(base + t * BCH, BCH)])
        rem = SL - nfull * BCH
        if rem:
            sync(vbuf.at[pl.ds(0, rem)], bacc.at[pl.ds(base + nfull * BCH, rem)])
        sync(a3_h, abuf)
        plsc.subcore_barrier()
        a3 = abuf[...]

        def chunk(j, carry):
            r0 = (w * NCHB + j) * 8
            sync(w_h.at[pl.ds(r0, 8)], wbuf)
            sync(dst_h.at[pl.ds(r0, 8)], dbuf)
            for r in range(8):
                for k in range(8):
                    v = wbuf[r, pl.ds(k * L, L)]
                    vbuf[pl.ds(r * 128 + k * L, L)] = jnp.maximum(a3 * v, 0.0)
            descs = [
                pltpu.async_copy(
                    vbuf.at[pl.ds(r * 128, 128)], bacc.at[dbuf.at[r]], sem, add=True
                )
                for r in range(8)
            ]
            for dsc in descs:
                dsc.wait()
            return carry

        lax.fori_loop(0, NCHB, chunk, 0)
        plsc.subcore_barrier()
        for t in range(nfull):
            sync(bacc.at[pl.ds(base + t * BCH, BCH)], vbuf)
            sync(vbuf, bp_h.at[pl.ds(c * N1 + base + t * BCH, BCH)])
        if rem:
            sync(bacc.at[pl.ds(base + nfull * BCH, rem)], vbuf.at[pl.ds(0, rem)])
            sync(vbuf.at[pl.ds(0, rem)], bp_h.at[pl.ds(c * N1 + base + nfull * BCH, rem)])

    @functools.partial(
        pl.kernel,
        out_type=jax.ShapeDtypeStruct((N1, D), f32),
        mesh=_mesh(),
        compiler_params=pltpu.CompilerParams(use_tc_tiling_on_sc=False),
        scratch_types=[
            pltpu.VMEM((RPT,), f32),
            pltpu.VMEM((RPT,), f32),
            pltpu.VMEM((RPT,), f32),
            pltpu.VMEM((L,), f32),
            pltpu.VMEM((L,), f32),
            pltpu.VMEM((RPT, D), f32),
        ],
    )
    def bias_rep(bp_h, lab_h, a2_h, a4_h, brep_h, b0, b1, lb, a2b, a4b, ob):
        w = _worker_id()
        base = w * RPT
        sync = pltpu.sync_copy
        sync(bp_h.at[pl.ds(base, RPT)], b0)
        sync(bp_h.at[pl.ds(N1 + base, RPT)], b1)
        sync(lab_h.at[pl.ds(base, RPT)], lb)
        sync(a2_h, a2b)
        sync(a4_h, a4b)
        a2 = a2b[...]
        a4 = a4b[...]

        def red(i, carry):
            o = i * L
            b0[pl.ds(o, L)] = a2 * (b0[pl.ds(o, L)] + b1[pl.ds(o, L)]) + a4 * lb[pl.ds(o, L)]
            return carry

        lax.fori_loop(0, RPT // L, red, 0)

        def bcast(i, carry):
            t16 = b0[pl.ds(i * L, L)]
            for k in range(L):
                t = jnp.full((L,), t16[k], f32)
                r = i * L + k
                for h in range(D // L):
                    ob[r, pl.ds(h * L, L)] = t
            return carry

        lax.fori_loop(0, RPT // L, bcast, 0)
        sync(ob, brep_h.at[pl.ds(base, RPT), :])

    SGC = 4
    NSG = NCH // SGC
    RB = CH // 128

    @functools.partial(
        pl.kernel,
        out_type=jax.ShapeDtypeStruct((NC * N1, D), f32),
        mesh=_mesh(),
        compiler_params=pltpu.CompilerParams(use_tc_tiling_on_sc=False),
        scratch_types=[
            pltpu.VMEM((SGC * RB, 128), i32),
            pltpu.VMEM((SGC * RB, 128), i32),
            pltpu.VMEM((2, CH, D), f32),
            pltpu.VMEM_SHARED((N1, D), f32),
            pltpu.SemaphoreType.DMA,
            pltpu.SemaphoreType.DMA,
            pltpu.SemaphoreType.DMA,
            pltpu.SemaphoreType.DMA,
        ],
    )
    def scatter_k(y_h, src_h, dst_h, pp_h, sstage, dstage, rows, acc, g0s, g1s, s0s, s1s):
        c = lax.axis_index("c")
        s = lax.axis_index("s")
        w = _worker_id()
        sync = pltpu.sync_copy
        gsems = (g0s, g1s)
        ssems = (s0s, s1s)

        def zr(r, carry):
            for h in range(D // L):
                rows[0, r, pl.ds(h * L, L)] = jnp.zeros((L,), f32)
            return carry

        lax.fori_loop(0, CH, zr, 0)
        base = s * SL
        nfull = SL // CH
        for t in range(nfull):
            sync(rows.at[0], acc.at[pl.ds(base + t * CH, CH), :])
        rem = SL - nfull * CH
        if rem:
            sync(rows.at[0, pl.ds(0, rem)], acc.at[pl.ds(base + nfull * CH, rem), :])
        plsc.subcore_barrier()

        def gather(q, slot):
            return [
                pltpu.async_copy(
                    y_h.at[sstage.at[q * RB + t]],
                    rows.at[slot, pl.ds(t * 128, 128)],
                    gsems[slot],
                )
                for t in range(RB)
            ]

        def scatter(q, slot):
            return [
                pltpu.async_copy(
                    rows.at[slot, pl.ds(t * 128, 128)],
                    acc.at[dstage.at[q * RB + t]],
                    ssems[slot],
                    add=True,
                )
                for t in range(RB)
            ]

        def wait(descs):
            for dsc in descs:
                dsc.wait()

        def sg(k, carry):
            r0 = (w * NSG + k) * (SGC * RB)
            sync(src_h.at[pl.ds(r0, SGC * RB)], sstage)
            sync(dst_h.at[pl.ds(r0, SGC * RB)], dstage)
            g0 = gather(0, 0)
            g1 = gather(1, 1)
            wait(g0)
            s0 = scatter(0, 0)
            wait(g1)
            s1 = scatter(1, 1)
            wait(s0)
            g2 = gather(2, 0)
            wait(s1)
            g3 = gather(3, 1)
            wait(g2)
            s2 = scatter(2, 0)
            wait(g3)
            s3 = scatter(3, 1)
            wait(s2)
            wait(s3)
            return carry

        lax.fori_loop(0, NSG, sg, 0)
        plsc.subcore_barrier()
        for t in range(nfull):
            sync(acc.at[pl.ds(base + t * CH, CH), :], rows.at[0])
            sync(rows.at[0], pp_h.at[pl.ds(c * N1 + base + t * CH, CH), :])
        if rem:
            sync(acc.at[pl.ds(base + nfull * CH, rem), :], rows.at[0, pl.ds(0, rem)])
            sync(rows.at[0, pl.ds(0, rem)], pp_h.at[pl.ds(c * N1 + base + nfull * CH, rem), :])

    CB = 512

    @functools.partial(
        pl.kernel,
        out_type=jax.ShapeDtypeStruct((N1, D), f32),
        mesh=_mesh(),
        compiler_params=pltpu.CompilerParams(use_tc_tiling_on_sc=False),
        scratch_types=[
            pltpu.VMEM((CB, D), f32),
            pltpu.VMEM((CB, D), f32),
            pltpu.VMEM((CB, D), f32),
            pltpu.VMEM((CB, D), f32),
            pltpu.VMEM((L,), f32),
        ],
    )
    def combine_k(pp_h, brep_h, a1_h, y_h, p0, p1, bb, ob, a1b):
        w = _worker_id()
        base = w * RPT
        sync = pltpu.sync_copy
        sync(a1_h, a1b)
        a1 = a1b[...]
        offs = [(o, min(CB, RPT - o)) for o in range(0, RPT, CB)]
        for o, n in offs:
            sync(pp_h.at[pl.ds(base + o, n), :], p0.at[pl.ds(0, n)])
            sync(pp_h.at[pl.ds(N1 + base + o, n), :], p1.at[pl.ds(0, n)])
            sync(brep_h.at[pl.ds(base + o, n), :], bb.at[pl.ds(0, n)])

            def row(r, carry):
                for h in range(D // L):
                    hs = pl.ds(h * L, L)
                    v = a1 * (p0[r, hs] + p1[r, hs]) + bb[r, hs]
                    ob[r, hs] = jnp.maximum(v, 0.0)
                return carry

            lax.fori_loop(0, n, row, 0)
            sync(ob.at[pl.ds(0, n)], y_h.at[pl.ds(base + o, n), :])

    return bias_partial, bias_rep, scatter_k, combine_k, N1, EP


def kernel(x, edge_index, edge_weight, labels, alpha1, alpha2, alpha3, alpha4):
    N, D = x.shape
    E = edge_index.shape[1]
    bias_partial, bias_rep, scatter_k, combine_k, N1, EP = _build(N, D, E)
    f32 = jnp.float32
    i32 = jnp.int32

    pe = EP - E
    src2 = jnp.concatenate([edge_index[0], jnp.zeros((pe,), i32)]).reshape(EP // 128, 128)
    dst2 = jnp.concatenate([edge_index[1], jnp.full((pe,), N, i32)]).reshape(EP // 128, 128)
    w2 = jnp.concatenate([edge_weight.astype(f32), jnp.zeros((pe,), f32)]).reshape(EP // 128, 128)
    xp = jnp.concatenate([x.astype(f32), jnp.zeros((N1 - N, D), f32)], axis=0)
    labp = jnp.concatenate([labels.astype(f32), jnp.zeros((N1 - N,), f32)])
    a1v = jnp.broadcast_to(alpha1.astype(f32), (L,))
    a2v = jnp.broadcast_to(alpha2.astype(f32), (L,))
    a3v = jnp.broadcast_to(alpha3.astype(f32), (L,))
    a4v = jnp.broadcast_to(alpha4.astype(f32), (L,))

    bp = bias_partial(w2, dst2, a3v)
    brep = bias_rep(bp, labp, a2v, a4v)
    y = xp
    for _ in range(4):
        pp = scatter_k(y, src2, dst2)
        y = combine_k(pp, brep, a1v)
    return y[:N]
